# baseline (speedup 1.0000x reference)
import functools

import numpy as np
import jax
import jax.numpy as jnp
from jax import lax
from jax.experimental import pallas as pl
from jax.experimental.pallas import tpu as pltpu
from jax.experimental.pallas import tpu_sc as plsc

_B = 4096
_EMB = 128
_L = 50
_BLK = 512

_N_SONGS_B = np.linspace(5.0, 500.0, 100).astype(np.float32)
_N_ART_B = np.linspace(1.0, 300.0, 100).astype(np.float32)
_N_ALB_B = np.linspace(1.0, 400.0, 100).astype(np.float32)
_DUR_B = np.linspace(30000.0, 600000.0, 100).astype(np.float32)
_APOP_B = np.linspace(0.0, 100.0, 10).astype(np.float32)
_FOLL_B = np.linspace(0.0, 1000000.0, 10).astype(np.float32)
_TPOP_B = np.linspace(0.0, 100.0, 10).astype(np.float32)


@functools.lru_cache(maxsize=None)
def _make_sc_kernel(n_plain, n_pooled):
    info = plsc.get_sparse_core_info()
    nc, ns = info.num_cores, info.num_subcores
    nw = nc * ns
    bpw = _B // nw

    mesh = plsc.VectorSubcoreMesh(core_axis_name="c", subcore_axis_name="s")
    out_type = tuple(
        jax.ShapeDtypeStruct((_B, _EMB), jnp.float32)
        for _ in range(n_plain + n_pooled))
    scratch = [
        pltpu.VMEM((bpw,), jnp.int32),
        pltpu.VMEM((bpw, _L), jnp.int32),
        pltpu.VMEM((bpw, _L), jnp.int32),
        pltpu.VMEM((8, _L, _EMB), jnp.float32),
        pltpu.VMEM((bpw, _EMB), jnp.float32),
        pltpu.VMEM((16, _EMB), jnp.float32),
        pltpu.VMEM_SHARED((ns * bpw, _EMB), jnp.float32),
    ] + [pltpu.SemaphoreType.DMA] * 16

    @functools.partial(pl.kernel, mesh=mesh, out_type=out_type,
                       scratch_types=scratch)
    def sc_k(*refs):
        nf = n_plain + n_pooled
        idxs = refs[:nf]
        tabs = refs[nf:2 * nf]
        outs = refs[2 * nf:3 * nf]
        (idx1_v, idxp_v, idxd_v, bufs, acc_v, zero_v, shacc,
         *sems) = refs[3 * nf:]
        sem_g = sems[:8]
        sem_s = sems[8:]
        sid = lax.axis_index("s")
        wid = sid * nc + lax.axis_index("c")
        base = wid * bpw
        sbase = sid * bpw

        def init_body(p, carry):
            slot = jnp.zeros((16,), jnp.int32) + (sbase + p)
            for o in (0, 16, 32, 34):
                idxd_v[p, pl.ds(o, 16)] = slot
            return carry

        lax.fori_loop(0, bpw, init_body, 0)

        def zinit_body(p, carry):
            z = jnp.zeros((16,), jnp.float32)
            for c in range(_EMB // 16):
                zero_v[p, pl.ds(c * 16, 16)] = z
            return carry

        lax.fori_loop(0, 16, zinit_body, 0)

        for ih, th, oh in zip(idxs[:n_plain], tabs[:n_plain], outs[:n_plain]):
            pltpu.sync_copy(ih.at[pl.ds(base, bpw)], idx1_v)
            pltpu.async_copy(th.at[idx1_v], acc_v, sem_g[0]).wait()
            pltpu.sync_copy(acc_v, oh.at[pl.ds(base, bpw)])

        for ih, th, oh in zip(idxs[n_plain:], tabs[n_plain:], outs[n_plain:]):
            pltpu.sync_copy(ih.at[pl.ds(base, bpw)], idxp_v)
            for z in range(bpw // 16):
                pltpu.sync_copy(zero_v, shacc.at[pl.ds(sbase + z * 16, 16)])
            for g in range(8):
                pltpu.async_copy(th.at[idxp_v.at[g]], bufs.at[g], sem_g[g])

            def body(i, carry, th=th):
                p0 = i * 8
                for g in range(8):
                    pltpu.make_async_copy(
                        th.at[idxp_v.at[p0 + g]], bufs.at[g], sem_g[g]).wait()
                    pltpu.async_copy(bufs.at[g], shacc.at[idxd_v.at[p0 + g]],
                                     sem_s[g], add=True)
                for g in range(8):
                    pltpu.make_async_copy(
                        bufs.at[g], shacc.at[idxd_v.at[p0 + g]],
                        sem_s[g]).wait()
                    pn = jnp.minimum(p0 + 8 + g, bpw - 1)
                    pltpu.async_copy(th.at[idxp_v.at[pn]], bufs.at[g],
                                     sem_g[g])
                return carry

            lax.fori_loop(0, bpw // 8, body, 0)
            for g in range(8):
                pltpu.make_async_copy(
                    th.at[idxp_v.at[bpw - 1]], bufs.at[g], sem_g[g]).wait()
            pltpu.sync_copy(shacc.at[pl.ds(sbase, bpw)], acc_v)
            pltpu.sync_copy(acc_v, oh.at[pl.ds(base, bpw)])

    return sc_k


def _hist_from_counts(c, total):
    blk = c.shape[0]
    left = jnp.concatenate(
        [jnp.full((blk, 1), total, jnp.float32), c], axis=1)
    right = jnp.concatenate([c, jnp.zeros((blk, 1), jnp.float32)], axis=1)
    return left - right


def _lanes(col, nb):
    return jnp.dot(col, jnp.ones((1, nb), jnp.float32),
                   preferred_element_type=jnp.float32,
                   precision=jax.lax.Precision.HIGHEST)


def _ge_counts(x, bins_row):
    nb = bins_row.shape[1]
    acc = (_lanes(x[:, 0:1], nb) >= bins_row).astype(jnp.float32)
    for j in range(1, x.shape[1]):
        acc = acc + (_lanes(x[:, j : j + 1], nb) >= bins_row).astype(
            jnp.float32)
    return acc


def _tc_body(nm, uc, s_an, s_tu, s_tn, s_al, s_ge,
             i_an, i_tu, i_tn, i_al, i_ge,
             collab, nsongs, nart, nalb, dur, apop, foll, tpop,
             e_collab, e_nsongs, e_nart, e_nalb, e_dur, e_apop, e_foll,
             e_tpop, r0_an, r0_tu, r0_tn, r0_al, r0_ge,
             bn_songs, bn_art, bn_alb, bn_dur, bn_apop, bn_foll, bn_tpop,
             cv, cu, cb, w1, b1, w2, b2, w3, b3, o_ref):
    f32 = jnp.float32

    def dot(a, b):
        return jnp.dot(a, b, preferred_element_type=f32)

    def onehot_feat(x_ref, bins_ref, tab_ref):
        nb = bins_ref.shape[1]
        c = (_lanes(x_ref[...], nb) >= bins_ref[...]).astype(f32)
        oh = _hist_from_counts(c, 1.0)
        return dot(oh, tab_ref[...])

    def masked_avg_feat(s_ref, i_ref, r0_ref):
        z = (i_ref[...] == 0).astype(f32)
        c0 = jnp.dot(z, jnp.ones((_L, 1), f32),
                     preferred_element_type=f32,
                     precision=jax.lax.Precision.HIGHEST)
        cnt = jnp.maximum(jnp.float32(_L) - c0, 1.0)
        return (s_ref[...] - _lanes(c0, _EMB) * r0_ref[...]) / _lanes(
            cnt, _EMB)

    def pooled_hist_feat(x_ref, bins_ref, tab_ref, masked):
        c = _ge_counts(x_ref[...], bins_ref[...])
        hist = _hist_from_counts(c, float(_L))
        s = dot(hist, tab_ref[...])
        if masked:
            h0 = hist[:, 0:1]
            s = s - _lanes(h0, _EMB) * tab_ref[0:1, :]
            return s / _lanes(jnp.maximum(jnp.float32(_L) - h0, 1.0), _EMB)
        return s * jnp.float32(1.0 / _L)

    coll = _lanes(collab[...].astype(f32), 3)
    thr = lax.broadcasted_iota(jnp.int32, (1, 3), 1).astype(f32) + 1.0
    c_coll = (coll >= thr).astype(f32)
    f_collab = dot(_hist_from_counts(c_coll, 1.0), e_collab[...])

    feats = [
        nm[...],
        f_collab,
        uc[...],
        onehot_feat(nsongs, bn_songs, e_nsongs),
        onehot_feat(nart, bn_art, e_nart),
        onehot_feat(nalb, bn_alb, e_nalb),
        masked_avg_feat(s_an, i_an, r0_an),
        masked_avg_feat(s_tu, i_tu, r0_tu),
        masked_avg_feat(s_tn, i_tn, r0_tn),
        pooled_hist_feat(dur, bn_dur, e_dur, masked=True),
        masked_avg_feat(s_al, i_al, r0_al),
        pooled_hist_feat(apop, bn_apop, e_apop, masked=False),
        pooled_hist_feat(foll, bn_foll, e_foll, masked=False),
        pooled_hist_feat(tpop, bn_tpop, e_tpop, masked=False),
        masked_avg_feat(s_ge, i_ge, r0_ge),
    ]
    x0 = jnp.concatenate(feats, axis=1)

    t = dot(x0, cv[...])
    s = dot(t, cu[...]) + cb[...]
    cross = x0 * s + x0
    h = jnp.maximum(dot(cross, w1[...]) + b1[...], 0.0)
    h = jnp.maximum(dot(h, w2[...]) + b2[...], 0.0)
    h = dot(h, w3[...]) + b3[...]
    ss = jnp.dot(h * h, jnp.ones((_EMB, 1), f32),
                 preferred_element_type=f32,
                 precision=jax.lax.Precision.HIGHEST)
    inv = 1.0 / jnp.maximum(jnp.sqrt(ss), 1e-12)
    o_ref[...] = h * _lanes(inv, _EMB)


def _tc_call(blocked, full):
    nblk = _B // _BLK

    def bspec(shape_b):
        return pl.BlockSpec(shape_b, lambda i: (i, 0))

    def fspec(arr):
        return pl.BlockSpec(arr.shape, lambda i: (0, 0))

    in_specs = (
        [bspec((_BLK, a.shape[1])) for a in blocked]
        + [fspec(a) for a in full]
    )
    return pl.pallas_call(
        _tc_body,
        grid=(nblk,),
        in_specs=in_specs,
        out_specs=pl.BlockSpec((_BLK, _EMB), lambda i: (i, 0)),
        out_shape=jax.ShapeDtypeStruct((_B, _EMB), jnp.float32),
    )(*blocked, *full)


def kernel(name_idx, collaborative_idx, track_uri_can_idx, artist_name_pl,
           track_uri_pl, track_name_pl, album_name_pl, artist_genres_pl,
           n_songs_pl, num_artists_pl, num_albums_pl, duration_ms_songs_pl,
           artist_pop_pl, artist_followers_pl, track_pop_pl, emb_name,
           emb_collab, emb_track_uri_can, emb_n_songs, emb_n_artists,
           emb_n_albums, emb_artist_name, emb_track_uri_pl, emb_track_name,
           emb_duration, emb_album_name, emb_artist_pop, emb_followers,
           emb_track_pop, emb_genres, cross_V, cross_U, cross_b, W1, b1, W2,
           b2, W3, b3):
    i32 = jnp.int32
    sc_k = _make_sc_kernel(2, 5)
    o_name, o_uri, o_an, o_tu, o_tn, o_al, o_ge = sc_k(
        name_idx.astype(i32), track_uri_can_idx.astype(i32),
        artist_name_pl.astype(i32), track_uri_pl.astype(i32),
        track_name_pl.astype(i32), album_name_pl.astype(i32),
        artist_genres_pl.astype(i32),
        emb_name, emb_track_uri_can, emb_artist_name, emb_track_uri_pl,
        emb_track_name, emb_album_name, emb_genres)

    blocked = [
        o_name, o_uri, o_an, o_tu, o_tn, o_al, o_ge,
        artist_name_pl.astype(i32), track_uri_pl.astype(i32),
        track_name_pl.astype(i32), album_name_pl.astype(i32),
        artist_genres_pl.astype(i32),
        collaborative_idx.astype(i32).reshape(_B, 1),
        n_songs_pl.reshape(_B, 1), num_artists_pl.reshape(_B, 1),
        num_albums_pl.reshape(_B, 1),
        duration_ms_songs_pl, artist_pop_pl, artist_followers_pl,
        track_pop_pl,
    ]
    full = [
        emb_collab, emb_n_songs, emb_n_artists, emb_n_albums, emb_duration,
        emb_artist_pop, emb_followers, emb_track_pop,
        emb_artist_name[0:1], emb_track_uri_pl[0:1], emb_track_name[0:1],
        emb_album_name[0:1], emb_genres[0:1],
        jnp.asarray(_N_SONGS_B).reshape(1, -1),
        jnp.asarray(_N_ART_B).reshape(1, -1),
        jnp.asarray(_N_ALB_B).reshape(1, -1),
        jnp.asarray(_DUR_B).reshape(1, -1),
        jnp.asarray(_APOP_B).reshape(1, -1),
        jnp.asarray(_FOLL_B).reshape(1, -1),
        jnp.asarray(_TPOP_B).reshape(1, -1),
        cross_V, cross_U, cross_b.reshape(1, -1),
        W1, b1.reshape(1, -1), W2, b2.reshape(1, -1), W3, b3.reshape(1, -1),
    ]
    return _tc_call(blocked, full)


# speedup vs baseline: 1.7022x; 1.7022x over previous
import functools

import numpy as np
import jax
import jax.numpy as jnp
from jax import lax
from jax.experimental import pallas as pl
from jax.experimental.pallas import tpu as pltpu
from jax.experimental.pallas import tpu_sc as plsc

_B = 4096
_EMB = 128
_L = 50
_BLK = 512

_N_SONGS_B = np.linspace(5.0, 500.0, 100).astype(np.float32)
_N_ART_B = np.linspace(1.0, 300.0, 100).astype(np.float32)
_N_ALB_B = np.linspace(1.0, 400.0, 100).astype(np.float32)
_DUR_B = np.linspace(30000.0, 600000.0, 100).astype(np.float32)
_APOP_B = np.linspace(0.0, 100.0, 10).astype(np.float32)
_FOLL_B = np.linspace(0.0, 1000000.0, 10).astype(np.float32)
_TPOP_B = np.linspace(0.0, 100.0, 10).astype(np.float32)


@functools.lru_cache(maxsize=None)
def _make_sc_kernel(n_plain, n_pooled):
    info = plsc.get_sparse_core_info()
    nc, ns = info.num_cores, info.num_subcores
    nw = nc * ns
    bpw = _B // nw

    mesh = plsc.VectorSubcoreMesh(core_axis_name="c", subcore_axis_name="s")
    out_type = tuple(
        jax.ShapeDtypeStruct((_B, _EMB), jnp.float32)
        for _ in range(n_plain + n_pooled))
    scratch = [
        pltpu.VMEM((bpw,), jnp.int32),
        pltpu.VMEM((bpw, _L), jnp.int32),
        pltpu.VMEM((bpw, _L), jnp.int32),
        pltpu.VMEM((8, _L, _EMB), jnp.float32),
        pltpu.VMEM((bpw, _EMB), jnp.float32),
        pltpu.VMEM((16, _EMB), jnp.float32),
        pltpu.VMEM_SHARED((ns * bpw, _EMB), jnp.float32),
    ] + [pltpu.SemaphoreType.DMA] * 16

    @functools.partial(pl.kernel, mesh=mesh, out_type=out_type,
                       scratch_types=scratch)
    def sc_k(*refs):
        nf = n_plain + n_pooled
        idxs = refs[:nf]
        tabs = refs[nf:2 * nf]
        outs = refs[2 * nf:3 * nf]
        (idx1_v, idxp_v, idxd_v, bufs, acc_v, zero_v, shacc,
         *sems) = refs[3 * nf:]
        sem_g = sems[:8]
        sem_s = sems[8:]
        sid = lax.axis_index("s")
        wid = sid * nc + lax.axis_index("c")
        base = wid * bpw
        sbase = sid * bpw

        def init_body(p, carry):
            slot = jnp.zeros((16,), jnp.int32) + (sbase + p)
            for o in (0, 16, 32, 34):
                idxd_v[p, pl.ds(o, 16)] = slot
            return carry

        lax.fori_loop(0, bpw, init_body, 0)

        def zinit_body(p, carry):
            z = jnp.zeros((16,), jnp.float32)
            for c in range(_EMB // 16):
                zero_v[p, pl.ds(c * 16, 16)] = z
            return carry

        lax.fori_loop(0, 16, zinit_body, 0)

        for ih, th, oh in zip(idxs[:n_plain], tabs[:n_plain], outs[:n_plain]):
            pltpu.sync_copy(ih.at[pl.ds(base, bpw)], idx1_v)
            pltpu.async_copy(th.at[idx1_v], acc_v, sem_g[0]).wait()
            pltpu.sync_copy(acc_v, oh.at[pl.ds(base, bpw)])

        for ih, th, oh in zip(idxs[n_plain:], tabs[n_plain:], outs[n_plain:]):
            pltpu.sync_copy(ih.at[pl.ds(base, bpw)], idxp_v)
            for z in range(bpw // 16):
                pltpu.sync_copy(zero_v, shacc.at[pl.ds(sbase + z * 16, 16)])
            for g in range(8):
                pltpu.async_copy(th.at[idxp_v.at[g]], bufs.at[g], sem_g[g])

            def body(i, carry, th=th):
                p0 = i * 8
                for g in range(8):
                    pltpu.make_async_copy(
                        th.at[idxp_v.at[p0 + g]], bufs.at[g], sem_g[g]).wait()
                    pltpu.async_copy(bufs.at[g], shacc.at[idxd_v.at[p0 + g]],
                                     sem_s[g], add=True)
                for g in range(8):
                    pltpu.make_async_copy(
                        bufs.at[g], shacc.at[idxd_v.at[p0 + g]],
                        sem_s[g]).wait()
                    pn = jnp.minimum(p0 + 8 + g, bpw - 1)
                    pltpu.async_copy(th.at[idxp_v.at[pn]], bufs.at[g],
                                     sem_g[g])
                return carry

            lax.fori_loop(0, bpw // 8, body, 0)
            for g in range(8):
                pltpu.make_async_copy(
                    th.at[idxp_v.at[bpw - 1]], bufs.at[g], sem_g[g]).wait()
            pltpu.sync_copy(shacc.at[pl.ds(sbase, bpw)], acc_v)
            pltpu.sync_copy(acc_v, oh.at[pl.ds(base, bpw)])

    return sc_k


def _hist_from_counts(c, total):
    blk = c.shape[0]
    left = jnp.concatenate(
        [jnp.full((blk, 1), total, jnp.float32), c], axis=1)
    right = jnp.concatenate([c, jnp.zeros((blk, 1), jnp.float32)], axis=1)
    return left - right


def _ge_counts(x, bins_row):
    acc = (x[:, 0:1] >= bins_row).astype(jnp.float32)
    for j in range(1, x.shape[1]):
        acc = acc + (x[:, j : j + 1] >= bins_row).astype(jnp.float32)
    return acc


def _tc_body(nm, uc, s_an, s_tu, s_tn, s_al, s_ge,
             i_an, i_tu, i_tn, i_al, i_ge,
             collab, nsongs, nart, nalb, dur, apop, foll, tpop,
             e_collab, e_nsongs, e_nart, e_nalb, e_dur, e_apop, e_foll,
             e_tpop, r0_an, r0_tu, r0_tn, r0_al, r0_ge,
             bn_songs, bn_art, bn_alb, bn_dur, bn_apop, bn_foll, bn_tpop,
             cv, cu, cb, w1, b1, w2, b2, w3, b3, o_ref):
    f32 = jnp.float32

    def dot(a, b):
        return jnp.dot(a, b, preferred_element_type=f32)

    def onehot_feat(x_ref, bins_ref, tab_ref):
        c = (x_ref[...] >= bins_ref[...]).astype(f32)
        oh = _hist_from_counts(c, 1.0)
        return dot(oh, tab_ref[...])

    def masked_avg_feat(s_ref, i_ref, r0_ref):
        c0 = jnp.sum((i_ref[...] == 0).astype(f32), axis=1, keepdims=True)
        cnt = jnp.maximum(jnp.float32(_L) - c0, 1.0)
        return (s_ref[...] - c0 * r0_ref[...]) / cnt

    def pooled_hist_feat(x_ref, bins_ref, tab_ref, masked):
        c = _ge_counts(x_ref[...], bins_ref[...])
        hist = _hist_from_counts(c, float(_L))
        s = dot(hist, tab_ref[...])
        if masked:
            h0 = hist[:, 0:1]
            s = s - h0 * tab_ref[0:1, :]
            return s / jnp.maximum(jnp.float32(_L) - h0, 1.0)
        return s * jnp.float32(1.0 / _L)

    coll = collab[...].astype(f32)
    thr = lax.broadcasted_iota(jnp.int32, (1, 3), 1).astype(f32) + 1.0
    c_coll = (coll >= thr).astype(f32)
    f_collab = dot(_hist_from_counts(c_coll, 1.0), e_collab[...])

    feats = [
        nm[...],
        f_collab,
        uc[...],
        onehot_feat(nsongs, bn_songs, e_nsongs),
        onehot_feat(nart, bn_art, e_nart),
        onehot_feat(nalb, bn_alb, e_nalb),
        masked_avg_feat(s_an, i_an, r0_an),
        masked_avg_feat(s_tu, i_tu, r0_tu),
        masked_avg_feat(s_tn, i_tn, r0_tn),
        pooled_hist_feat(dur, bn_dur, e_dur, masked=True),
        masked_avg_feat(s_al, i_al, r0_al),
        pooled_hist_feat(apop, bn_apop, e_apop, masked=False),
        pooled_hist_feat(foll, bn_foll, e_foll, masked=False),
        pooled_hist_feat(tpop, bn_tpop, e_tpop, masked=False),
        masked_avg_feat(s_ge, i_ge, r0_ge),
    ]
    x0 = jnp.concatenate(feats, axis=1)

    t = dot(x0, cv[...])
    s = dot(t, cu[...]) + cb[...]
    cross = x0 * s + x0
    h = jnp.maximum(dot(cross, w1[...]) + b1[...], 0.0)
    h = jnp.maximum(dot(h, w2[...]) + b2[...], 0.0)
    h = dot(h, w3[...]) + b3[...]
    ss = jnp.sum(h * h, axis=1, keepdims=True)
    o_ref[...] = h / jnp.maximum(jnp.sqrt(ss), 1e-12)


def _tc_call(blocked, full):
    nblk = _B // _BLK

    def bspec(shape_b):
        return pl.BlockSpec(shape_b, lambda i: (i, 0))

    def fspec(arr):
        return pl.BlockSpec(arr.shape, lambda i: (0, 0))

    in_specs = (
        [bspec((_BLK, a.shape[1])) for a in blocked]
        + [fspec(a) for a in full]
    )
    return pl.pallas_call(
        _tc_body,
        grid=(nblk,),
        in_specs=in_specs,
        out_specs=pl.BlockSpec((_BLK, _EMB), lambda i: (i, 0)),
        out_shape=jax.ShapeDtypeStruct((_B, _EMB), jnp.float32),
    )(*blocked, *full)


def kernel(name_idx, collaborative_idx, track_uri_can_idx, artist_name_pl,
           track_uri_pl, track_name_pl, album_name_pl, artist_genres_pl,
           n_songs_pl, num_artists_pl, num_albums_pl, duration_ms_songs_pl,
           artist_pop_pl, artist_followers_pl, track_pop_pl, emb_name,
           emb_collab, emb_track_uri_can, emb_n_songs, emb_n_artists,
           emb_n_albums, emb_artist_name, emb_track_uri_pl, emb_track_name,
           emb_duration, emb_album_name, emb_artist_pop, emb_followers,
           emb_track_pop, emb_genres, cross_V, cross_U, cross_b, W1, b1, W2,
           b2, W3, b3):
    i32 = jnp.int32
    sc_k = _make_sc_kernel(2, 5)
    o_name, o_uri, o_an, o_tu, o_tn, o_al, o_ge = sc_k(
        name_idx.astype(i32), track_uri_can_idx.astype(i32),
        artist_name_pl.astype(i32), track_uri_pl.astype(i32),
        track_name_pl.astype(i32), album_name_pl.astype(i32),
        artist_genres_pl.astype(i32),
        emb_name, emb_track_uri_can, emb_artist_name, emb_track_uri_pl,
        emb_track_name, emb_album_name, emb_genres)

    blocked = [
        o_name, o_uri, o_an, o_tu, o_tn, o_al, o_ge,
        artist_name_pl.astype(i32), track_uri_pl.astype(i32),
        track_name_pl.astype(i32), album_name_pl.astype(i32),
        artist_genres_pl.astype(i32),
        collaborative_idx.astype(i32).reshape(_B, 1),
        n_songs_pl.reshape(_B, 1), num_artists_pl.reshape(_B, 1),
        num_albums_pl.reshape(_B, 1),
        duration_ms_songs_pl, artist_pop_pl, artist_followers_pl,
        track_pop_pl,
    ]
    full = [
        emb_collab, emb_n_songs, emb_n_artists, emb_n_albums, emb_duration,
        emb_artist_pop, emb_followers, emb_track_pop,
        emb_artist_name[0:1], emb_track_uri_pl[0:1], emb_track_name[0:1],
        emb_album_name[0:1], emb_genres[0:1],
        jnp.asarray(_N_SONGS_B).reshape(1, -1),
        jnp.asarray(_N_ART_B).reshape(1, -1),
        jnp.asarray(_N_ALB_B).reshape(1, -1),
        jnp.asarray(_DUR_B).reshape(1, -1),
        jnp.asarray(_APOP_B).reshape(1, -1),
        jnp.asarray(_FOLL_B).reshape(1, -1),
        jnp.asarray(_TPOP_B).reshape(1, -1),
        cross_V, cross_U, cross_b.reshape(1, -1),
        W1, b1.reshape(1, -1), W2, b2.reshape(1, -1), W3, b3.reshape(1, -1),
    ]
    return _tc_call(blocked, full)


# speedup vs baseline: 2.2206x; 1.3045x over previous
import functools

import numpy as np
import jax
import jax.numpy as jnp
from jax import lax
from jax.experimental import pallas as pl
from jax.experimental.pallas import tpu as pltpu
from jax.experimental.pallas import tpu_sc as plsc

_B = 4096
_EMB = 128
_L = 50
_BLK = 512

_N_SONGS_B = np.linspace(5.0, 500.0, 100).astype(np.float32)
_N_ART_B = np.linspace(1.0, 300.0, 100).astype(np.float32)
_N_ALB_B = np.linspace(1.0, 400.0, 100).astype(np.float32)
_DUR_B = np.linspace(30000.0, 600000.0, 100).astype(np.float32)
_APOP_B = np.linspace(0.0, 100.0, 10).astype(np.float32)
_FOLL_B = np.linspace(0.0, 1000000.0, 10).astype(np.float32)
_TPOP_B = np.linspace(0.0, 100.0, 10).astype(np.float32)


@functools.lru_cache(maxsize=None)
def _make_sc_kernel(n_plain, n_pooled):
    info = plsc.get_sparse_core_info()
    nc, ns = info.num_cores, info.num_subcores
    nw = nc * ns
    bpw = _B // nw

    mesh = plsc.VectorSubcoreMesh(core_axis_name="c", subcore_axis_name="s")
    out_type = tuple(
        jax.ShapeDtypeStruct((_B, _EMB), jnp.float32)
        for _ in range(n_plain + n_pooled))
    scratch = [
        pltpu.VMEM((bpw,), jnp.int32),
        pltpu.VMEM((bpw, _L), jnp.int32),
        pltpu.VMEM((bpw, _L), jnp.int32),
        pltpu.VMEM((8, _L, _EMB), jnp.float32),
        pltpu.VMEM((bpw, _EMB), jnp.float32),
        pltpu.VMEM((16, _EMB), jnp.float32),
        pltpu.VMEM_SHARED((ns * bpw, _EMB), jnp.float32),
    ] + [pltpu.SemaphoreType.DMA] * 16

    @functools.partial(pl.kernel, mesh=mesh, out_type=out_type,
                       scratch_types=scratch)
    def sc_k(*refs):
        nf = n_plain + n_pooled
        idxs = refs[:nf]
        tabs = refs[nf:2 * nf]
        outs = refs[2 * nf:3 * nf]
        (idx1_v, idxp_v, idxd_v, bufs, acc_v, zero_v, shacc,
         *sems) = refs[3 * nf:]
        sem_g = sems[:8]
        sem_s = sems[8:]
        sid = lax.axis_index("s")
        wid = sid * nc + lax.axis_index("c")
        base = wid * bpw
        sbase = sid * bpw

        def init_body(p, carry):
            slot = jnp.zeros((16,), jnp.int32) + (sbase + p)
            for o in (0, 16, 32, 34):
                idxd_v[p, pl.ds(o, 16)] = slot
            return carry

        lax.fori_loop(0, bpw, init_body, 0)

        def zinit_body(p, carry):
            z = jnp.zeros((16,), jnp.float32)
            for c in range(_EMB // 16):
                zero_v[p, pl.ds(c * 16, 16)] = z
            return carry

        lax.fori_loop(0, 16, zinit_body, 0)

        for ih, th, oh in zip(idxs[:n_plain], tabs[:n_plain], outs[:n_plain]):
            pltpu.sync_copy(ih.at[pl.ds(base, bpw)], idx1_v)
            pltpu.async_copy(th.at[idx1_v], acc_v, sem_g[0]).wait()
            pltpu.sync_copy(acc_v, oh.at[pl.ds(base, bpw)])

        for ih, th, oh in zip(idxs[n_plain:], tabs[n_plain:], outs[n_plain:]):
            pltpu.sync_copy(ih.at[pl.ds(base, bpw)], idxp_v)
            for z in range(bpw // 16):
                pltpu.sync_copy(zero_v, shacc.at[pl.ds(sbase + z * 16, 16)])
            for g in range(8):
                pltpu.async_copy(th.at[idxp_v.at[g]], bufs.at[g], sem_g[g])

            def body(i, carry, th=th):
                p0 = i * 8
                for g in range(8):
                    pltpu.make_async_copy(
                        th.at[idxp_v.at[p0 + g]], bufs.at[g], sem_g[g]).wait()
                    pltpu.async_copy(bufs.at[g], shacc.at[idxd_v.at[p0 + g]],
                                     sem_s[g], add=True)
                for g in range(8):
                    pltpu.make_async_copy(
                        bufs.at[g], shacc.at[idxd_v.at[p0 + g]],
                        sem_s[g]).wait()
                    pn = jnp.minimum(p0 + 8 + g, bpw - 1)
                    pltpu.async_copy(th.at[idxp_v.at[pn]], bufs.at[g],
                                     sem_g[g])
                return carry

            lax.fori_loop(0, bpw // 8, body, 0)
            for g in range(8):
                pltpu.make_async_copy(
                    th.at[idxp_v.at[bpw - 1]], bufs.at[g], sem_g[g]).wait()
            pltpu.sync_copy(shacc.at[pl.ds(sbase, bpw)], acc_v)
            pltpu.sync_copy(acc_v, oh.at[pl.ds(base, bpw)])

    return sc_k


def _hist_from_counts(c, total):
    blk = c.shape[0]
    left = jnp.concatenate(
        [jnp.full((blk, 1), total, jnp.float32), c], axis=1)
    right = jnp.concatenate([c, jnp.zeros((blk, 1), jnp.float32)], axis=1)
    return left - right


def _ge_counts(x, bins_row):
    acc = (x[:, 0:1] >= bins_row).astype(jnp.float32)
    for j in range(1, x.shape[1]):
        acc = acc + (x[:, j : j + 1] >= bins_row).astype(jnp.float32)
    return acc


def _tca_body(collab, nsongs, nart, nalb, dur, apop, foll, tpop,
              e_collab, e_nsongs, e_nart, e_nalb, e_dur, e_apop, e_foll,
              e_tpop,
              bn_songs, bn_art, bn_alb, bn_dur, bn_apop, bn_foll, bn_tpop,
              o_ref):
    f32 = jnp.float32

    def dot(a, b):
        return jnp.dot(a, b, preferred_element_type=f32)

    def onehot_feat(x_ref, bins_ref, tab_ref):
        c = (x_ref[...] >= bins_ref[...]).astype(f32)
        oh = _hist_from_counts(c, 1.0)
        return dot(oh, tab_ref[...])

    def pooled_hist_feat(x_ref, bins_ref, tab_ref, masked):
        c = _ge_counts(x_ref[...], bins_ref[...])
        hist = _hist_from_counts(c, float(_L))
        s = dot(hist, tab_ref[...])
        if masked:
            h0 = hist[:, 0:1]
            s = s - h0 * tab_ref[0:1, :]
            return s / jnp.maximum(jnp.float32(_L) - h0, 1.0)
        return s * jnp.float32(1.0 / _L)

    coll = collab[...].astype(f32)
    thr = lax.broadcasted_iota(jnp.int32, (1, 3), 1).astype(f32) + 1.0
    c_coll = (coll >= thr).astype(f32)
    f_collab = dot(_hist_from_counts(c_coll, 1.0), e_collab[...])

    o_ref[...] = jnp.concatenate([
        f_collab,
        onehot_feat(nsongs, bn_songs, e_nsongs),
        onehot_feat(nart, bn_art, e_nart),
        onehot_feat(nalb, bn_alb, e_nalb),
        pooled_hist_feat(dur, bn_dur, e_dur, masked=True),
        pooled_hist_feat(apop, bn_apop, e_apop, masked=False),
        pooled_hist_feat(foll, bn_foll, e_foll, masked=False),
        pooled_hist_feat(tpop, bn_tpop, e_tpop, masked=False),
    ], axis=1)


def _tcb_body(nm, uc, s_an, s_tu, s_tn, s_al, s_ge,
              i_an, i_tu, i_tn, i_al, i_ge, pa,
              r0_an, r0_tu, r0_tn, r0_al, r0_ge,
              cv, cu, cb, w1, b1, w2, b2, w3, b3, o_ref):
    f32 = jnp.float32

    def dot(a, b):
        return jnp.dot(a, b, preferred_element_type=f32)

    def masked_avg_feat(s_ref, i_ref, r0_ref):
        c0 = jnp.sum((i_ref[...] == 0).astype(f32), axis=1, keepdims=True)
        cnt = jnp.maximum(jnp.float32(_L) - c0, 1.0)
        return (s_ref[...] - c0 * r0_ref[...]) / cnt

    a = pa[...]
    e = _EMB
    feats = [
        nm[...],
        a[:, 0:e],
        uc[...],
        a[:, e:4 * e],
        masked_avg_feat(s_an, i_an, r0_an),
        masked_avg_feat(s_tu, i_tu, r0_tu),
        masked_avg_feat(s_tn, i_tn, r0_tn),
        a[:, 4 * e:5 * e],
        masked_avg_feat(s_al, i_al, r0_al),
        a[:, 5 * e:8 * e],
        masked_avg_feat(s_ge, i_ge, r0_ge),
    ]
    x0 = jnp.concatenate(feats, axis=1)

    t = dot(x0, cv[...])
    s = dot(t, cu[...]) + cb[...]
    cross = x0 * s + x0
    h = jnp.maximum(dot(cross, w1[...]) + b1[...], 0.0)
    h = jnp.maximum(dot(h, w2[...]) + b2[...], 0.0)
    h = dot(h, w3[...]) + b3[...]
    ss = jnp.sum(h * h, axis=1, keepdims=True)
    o_ref[...] = h / jnp.maximum(jnp.sqrt(ss), 1e-12)


def _tc_call(body, blocked, full, out_cols):
    nblk = _B // _BLK

    def bspec(shape_b):
        return pl.BlockSpec(shape_b, lambda i: (i, 0))

    def fspec(arr):
        return pl.BlockSpec(arr.shape, lambda i: (0, 0))

    in_specs = (
        [bspec((_BLK, a.shape[1])) for a in blocked]
        + [fspec(a) for a in full]
    )
    return pl.pallas_call(
        body,
        grid=(nblk,),
        in_specs=in_specs,
        out_specs=pl.BlockSpec((_BLK, out_cols), lambda i: (i, 0)),
        out_shape=jax.ShapeDtypeStruct((_B, out_cols), jnp.float32),
    )(*blocked, *full)


def kernel(name_idx, collaborative_idx, track_uri_can_idx, artist_name_pl,
           track_uri_pl, track_name_pl, album_name_pl, artist_genres_pl,
           n_songs_pl, num_artists_pl, num_albums_pl, duration_ms_songs_pl,
           artist_pop_pl, artist_followers_pl, track_pop_pl, emb_name,
           emb_collab, emb_track_uri_can, emb_n_songs, emb_n_artists,
           emb_n_albums, emb_artist_name, emb_track_uri_pl, emb_track_name,
           emb_duration, emb_album_name, emb_artist_pop, emb_followers,
           emb_track_pop, emb_genres, cross_V, cross_U, cross_b, W1, b1, W2,
           b2, W3, b3):
    i32 = jnp.int32
    sc_k = _make_sc_kernel(2, 5)
    o_name, o_uri, o_an, o_tu, o_tn, o_al, o_ge = sc_k(
        name_idx.astype(i32), track_uri_can_idx.astype(i32),
        artist_name_pl.astype(i32), track_uri_pl.astype(i32),
        track_name_pl.astype(i32), album_name_pl.astype(i32),
        artist_genres_pl.astype(i32),
        emb_name, emb_track_uri_can, emb_artist_name, emb_track_uri_pl,
        emb_track_name, emb_album_name, emb_genres)

    blocked_a = [
        collaborative_idx.astype(i32).reshape(_B, 1),
        n_songs_pl.reshape(_B, 1), num_artists_pl.reshape(_B, 1),
        num_albums_pl.reshape(_B, 1),
        duration_ms_songs_pl, artist_pop_pl, artist_followers_pl,
        track_pop_pl,
    ]
    full_a = [
        emb_collab, emb_n_songs, emb_n_artists, emb_n_albums, emb_duration,
        emb_artist_pop, emb_followers, emb_track_pop,
        jnp.asarray(_N_SONGS_B).reshape(1, -1),
        jnp.asarray(_N_ART_B).reshape(1, -1),
        jnp.asarray(_N_ALB_B).reshape(1, -1),
        jnp.asarray(_DUR_B).reshape(1, -1),
        jnp.asarray(_APOP_B).reshape(1, -1),
        jnp.asarray(_FOLL_B).reshape(1, -1),
        jnp.asarray(_TPOP_B).reshape(1, -1),
    ]
    part_a = _tc_call(_tca_body, blocked_a, full_a, 8 * _EMB)

    blocked_b = [
        o_name, o_uri, o_an, o_tu, o_tn, o_al, o_ge,
        artist_name_pl.astype(i32), track_uri_pl.astype(i32),
        track_name_pl.astype(i32), album_name_pl.astype(i32),
        artist_genres_pl.astype(i32),
        part_a,
    ]
    full_b = [
        emb_artist_name[0:1], emb_track_uri_pl[0:1], emb_track_name[0:1],
        emb_album_name[0:1], emb_genres[0:1],
        cross_V, cross_U, cross_b.reshape(1, -1),
        W1, b1.reshape(1, -1), W2, b2.reshape(1, -1), W3, b3.reshape(1, -1),
    ]
    return _tc_call(_tcb_body, blocked_b, full_b, _EMB)
